# Initial kernel scaffold; baseline (speedup 1.0000x reference)
import jax
import jax.numpy as jnp
from jax.experimental import pallas as pl


def kernel(X, expert_W, expert_b, router_W, router_b, out_W, out_b):
    raise NotImplementedError("write your pallas kernel here")


# speedup vs baseline: 1.5056x; 1.5056x over previous
import functools

import jax
import jax.numpy as jnp
from jax.experimental import pallas as pl

B = 2
S = 2048
D = 1024
E = 8
TOP_K = 2
TILE = 512
N_TOKENS = B * S


def _moe_kernel(x_ref, ewt_ref, eb_ref, rwt_ref, rb_ref, owt_ref, ob_ref,
                o_ref):
    x = x_ref[...]
    xb = x.astype(jnp.bfloat16)

    logits = jax.lax.dot_general(
        xb, rwt_ref[...],
        dimension_numbers=(((1,), (0,)), ((), ())),
        preferred_element_type=jnp.float32) + rb_ref[...]

    m = jnp.max(logits, axis=-1, keepdims=True)
    ex = jnp.exp(logits - m)
    w = ex / jnp.sum(ex, axis=-1, keepdims=True)

    iota = jax.lax.broadcasted_iota(jnp.int32, (TILE, E), 1)
    m1 = jnp.max(w, axis=-1, keepdims=True)
    e1 = jnp.min(jnp.where(w == m1, iota, E), axis=-1, keepdims=True)
    sel1 = iota == e1
    w2 = jnp.where(sel1, -jnp.inf, w)
    m2 = jnp.max(w2, axis=-1, keepdims=True)
    e2 = jnp.min(jnp.where(w2 == m2, iota, E), axis=-1, keepdims=True)
    sel2 = iota == e2
    scale = jnp.where(sel1 | sel2, w, 0.0)

    y = jax.lax.dot_general(
        xb, ewt_ref[...],
        dimension_numbers=(((1,), (0,)), ((), ())),
        preferred_element_type=jnp.float32)

    acc = jnp.zeros((TILE, D), jnp.float32)
    for e in range(E):
        se = jax.lax.slice(scale, (0, e), (TILE, e + 1))
        ye = y[:, e * D:(e + 1) * D] + eb_ref[e, :][None, :]
        acc = acc + se * ye

    out = jax.lax.dot_general(
        acc.astype(jnp.bfloat16), owt_ref[...],
        dimension_numbers=(((1,), (0,)), ((), ())),
        preferred_element_type=jnp.float32) + ob_ref[...]
    o_ref[...] = out


@functools.partial(jax.jit, static_argnames=())
def kernel(X, expert_W, expert_b, router_W, router_b, out_W, out_b):
    x2d = X.reshape(N_TOKENS, D)
    ewt = jnp.transpose(expert_W, (2, 0, 1)).reshape(D, E * D)
    ewt = ewt.astype(jnp.bfloat16)
    owt = out_W.T.astype(jnp.bfloat16)
    rwt = router_W.T.astype(jnp.bfloat16)
    rb = router_b.reshape(1, E)
    ob = out_b.reshape(1, D)

    grid = (N_TOKENS // TILE,)
    out = pl.pallas_call(
        _moe_kernel,
        grid=grid,
        in_specs=[
            pl.BlockSpec((TILE, D), lambda i: (i, 0)),
            pl.BlockSpec((D, E * D), lambda i: (0, 0)),
            pl.BlockSpec((E, D), lambda i: (0, 0)),
            pl.BlockSpec((D, E), lambda i: (0, 0)),
            pl.BlockSpec((1, E), lambda i: (0, 0)),
            pl.BlockSpec((D, D), lambda i: (0, 0)),
            pl.BlockSpec((1, D), lambda i: (0, 0)),
        ],
        out_specs=pl.BlockSpec((TILE, D), lambda i: (i, 0)),
        out_shape=jax.ShapeDtypeStruct((N_TOKENS, D), jnp.float32),
    )(x2d, ewt, expert_b, rwt, rb, owt, ob)
    return out.reshape(B, S, D)


# speedup vs baseline: 2.0827x; 1.3833x over previous
import jax
import jax.numpy as jnp
from jax.experimental import pallas as pl

B = 2
S = 2048
D = 1024
E = 8
TOP_K = 2
TILE = 512
N_TOKENS = B * S


def _moe_kernel(x_ref, ew_ref, eb_ref, rwt_ref, rb_ref, ow_ref, ob_ref,
                o_ref):
    x = x_ref[...]
    xb = x.astype(jnp.bfloat16)

    logits = jax.lax.dot_general(
        xb, rwt_ref[...],
        dimension_numbers=(((1,), (0,)), ((), ())),
        preferred_element_type=jnp.float32) + rb_ref[...]

    m = jnp.max(logits, axis=-1, keepdims=True)
    ex = jnp.exp(logits - m)
    w = ex / jnp.sum(ex, axis=-1, keepdims=True)

    iota = jax.lax.broadcasted_iota(jnp.int32, (TILE, E), 1)
    m1 = jnp.max(w, axis=-1, keepdims=True)
    e1 = jnp.min(jnp.where(w == m1, iota, E), axis=-1, keepdims=True)
    sel1 = iota == e1
    w2 = jnp.where(sel1, -jnp.inf, w)
    m2 = jnp.max(w2, axis=-1, keepdims=True)
    e2 = jnp.min(jnp.where(w2 == m2, iota, E), axis=-1, keepdims=True)
    sel2 = iota == e2
    scale = jnp.where(sel1 | sel2, w, 0.0)

    acc = jnp.zeros((TILE, D), jnp.float32)
    for e in range(E):
        we = ew_ref[e].astype(jnp.bfloat16)
        y = jax.lax.dot_general(
            xb, we,
            dimension_numbers=(((1,), (1,)), ((), ())),
            preferred_element_type=jnp.float32)
        se = jax.lax.slice(scale, (0, e), (TILE, e + 1))
        acc = acc + se * (y + eb_ref[e, :][None, :])

    out = jax.lax.dot_general(
        acc.astype(jnp.bfloat16), ow_ref[...].astype(jnp.bfloat16),
        dimension_numbers=(((1,), (1,)), ((), ())),
        preferred_element_type=jnp.float32) + ob_ref[...]
    o_ref[...] = out


def kernel(X, expert_W, expert_b, router_W, router_b, out_W, out_b):
    x2d = X.reshape(N_TOKENS, D)
    rwt = router_W.T.astype(jnp.bfloat16)
    rb = router_b.reshape(1, E)
    ob = out_b.reshape(1, D)

    out = pl.pallas_call(
        _moe_kernel,
        grid=(N_TOKENS // TILE,),
        in_specs=[
            pl.BlockSpec((TILE, D), lambda i: (i, 0)),
            pl.BlockSpec((E, D, D), lambda i: (0, 0, 0)),
            pl.BlockSpec((E, D), lambda i: (0, 0)),
            pl.BlockSpec((D, E), lambda i: (0, 0)),
            pl.BlockSpec((1, E), lambda i: (0, 0)),
            pl.BlockSpec((D, D), lambda i: (0, 0)),
            pl.BlockSpec((1, D), lambda i: (0, 0)),
        ],
        out_specs=pl.BlockSpec((TILE, D), lambda i: (i, 0)),
        out_shape=jax.ShapeDtypeStruct((N_TOKENS, D), jnp.float32),
    )(x2d, expert_W, expert_b, rwt, rb, out_W, ob)
    return out.reshape(B, S, D)
